# baseline (speedup 1.0000x reference)
import functools

import jax
import jax.numpy as jnp
import numpy as np
from jax import lax
from jax.experimental import pallas as pl
from jax.experimental.pallas import tpu as pltpu
from jax.experimental.pallas import tpu_sc as plsc

F = 26
E = 32
B = 4096
H0, H1, H2 = 256, 128, 64
V = 100000
RING = 28
ND = 14
KPAD = RING * E
NROWS = B * F


def _pair_perm():
    def pair_index(a, b):
        return a * (2 * F - a - 1) // 2 + (b - a - 1)

    pid = np.zeros((ND, RING), dtype=np.int32)
    scale = np.zeros((ND, RING), dtype=np.float32)
    for d in range(1, ND + 1):
        for f in range(RING):
            i, j = f, (f + d) % RING
            if i < F and j < F and i != j:
                pid[d - 1, f] = pair_index(min(i, j), max(i, j))
                scale[d - 1, f] = 0.5 if d == ND else 1.0
    return pid.reshape(-1), scale.reshape(-1)


_PID, _SCALE = _pair_perm()


_NW = 32
_RPW = NROWS // _NW
_CH = 128
_NCH = _RPW // _CH


def _sc_gather_body(idx_hbm, tab_hbm, out_hbm, idx_v, rows_v, sem):
    wid = lax.axis_index("s") * 2 + lax.axis_index("c")
    base = wid * _RPW
    pltpu.sync_copy(idx_hbm.at[pl.ds(base, _RPW)], idx_v)
    for j in range(_NCH):
        pltpu.async_copy(tab_hbm.at[idx_v.at[pl.ds(j * _CH, _CH)]],
                         rows_v.at[pl.ds(j * _CH, _CH)], sem)
    for j in range(_NCH):
        pltpu.make_async_copy(tab_hbm.at[idx_v.at[pl.ds(j * _CH, _CH)]],
                              rows_v.at[pl.ds(j * _CH, _CH)], sem).wait()
    pltpu.sync_copy(rows_v, out_hbm.at[pl.ds(base, _RPW)])


def _sc_gather(flat_idx, flat_tables):
    mesh = plsc.VectorSubcoreMesh(core_axis_name="c", subcore_axis_name="s")
    k = pl.kernel(
        _sc_gather_body,
        mesh=mesh,
        compiler_params=pltpu.CompilerParams(use_tc_tiling_on_sc=False),
        out_type=jax.ShapeDtypeStruct((NROWS, E), jnp.float32),
        scratch_types=[
            pltpu.VMEM((_RPW,), jnp.int32),
            pltpu.VMEM((_RPW, E), jnp.float32),
            pltpu.SemaphoreType.DMA,
        ],
    )
    return k(flat_idx, flat_tables)


_BT = 256
_GRID = B // _BT


def _tc_body(e_ref, wbig_ref, lb_ref, w1_ref, b1_ref, w2_ref, b2_ref,
             wf_ref, bf_ref, out_ref):
    ep = jnp.concatenate(
        [e_ref[...], jnp.zeros((_BT, KPAD - F * E), jnp.float32)], axis=1)
    acc = jnp.dot(ep, wbig_ref[0:KPAD, :], preferred_element_type=jnp.float32)
    for d in range(1, ND + 1):
        s = E * d
        rot = jnp.concatenate([ep[:, s:], ep[:, :s]], axis=1)
        acc += jnp.dot(ep * rot, wbig_ref[d * KPAD:(d + 1) * KPAD, :],
                       preferred_element_type=jnp.float32)
    x = jnp.maximum(acc + lb_ref[...], 0.0)
    x = jnp.maximum(jnp.dot(x, w1_ref[...],
                            preferred_element_type=jnp.float32) + b1_ref[...], 0.0)
    x = jnp.maximum(jnp.dot(x, w2_ref[...],
                            preferred_element_type=jnp.float32) + b2_ref[...], 0.0)
    z = jnp.dot(x, wf_ref[...], preferred_element_type=jnp.float32) + bf_ref[...]
    out_ref[...] = 1.0 / (1.0 + jnp.exp(-z))


def _prep_wbig(w_z, w_p):
    wz = w_z.reshape(F * E, H0)
    wz = jnp.concatenate([wz, jnp.zeros((KPAD - F * E, H0), jnp.float32)], axis=0)
    wp = jnp.take(w_p, jnp.asarray(_PID), axis=0)
    wp = wp * jnp.asarray(_SCALE)[:, None, None]
    return jnp.concatenate([wz, wp.reshape(ND * KPAD, H0)], axis=0)


def _tc_call(e2, w_big, l_b, W1, b1, W2, b2, Wf, bf):
    const = lambda i: (0, 0)
    return pl.pallas_call(
        _tc_body,
        grid=(_GRID,),
        in_specs=[
            pl.BlockSpec((_BT, F * E), lambda i: (i, 0)),
            pl.BlockSpec(((ND + 1) * KPAD, H0), const),
            pl.BlockSpec((1, H0), const),
            pl.BlockSpec((H0, H1), const),
            pl.BlockSpec((1, H1), const),
            pl.BlockSpec((H1, H2), const),
            pl.BlockSpec((1, H2), const),
            pl.BlockSpec((H2, 1), const),
            pl.BlockSpec((1, 1), const),
        ],
        out_specs=pl.BlockSpec((_BT, 1), lambda i: (i, 0)),
        out_shape=jax.ShapeDtypeStruct((B, 1), jnp.float32),
    )(e2, w_big, l_b.reshape(1, H0), W1, b1.reshape(1, H1),
      W2, b2.reshape(1, H2), Wf, bf.reshape(1, 1))


def kernel(indices, tables, w_z, w_p, l_b, W1, b1, W2, b2, Wf, bf):
    flat_idx = (indices + V * jnp.arange(F, dtype=jnp.int32)[None, :])
    flat_idx = flat_idx.reshape(NROWS)
    e = _sc_gather(flat_idx, tables.reshape(F * V, E))
    e2 = e.reshape(B, F * E)
    w_big = _prep_wbig(w_z, w_p)
    return _tc_call(e2, w_big, l_b, W1, b1, W2, b2, Wf, bf)


# speedup vs baseline: 1.6886x; 1.6886x over previous
import functools

import jax
import jax.numpy as jnp
import numpy as np
from jax import lax
from jax.experimental import pallas as pl
from jax.experimental.pallas import tpu as pltpu
from jax.experimental.pallas import tpu_sc as plsc

F = 26
E = 32
B = 4096
H0, H1, H2 = 256, 128, 64
V = 100000
RING = 28
ND = 14
KPAD = RING * E
NROWS = B * F


def _pair_perm():
    def pair_index(a, b):
        return a * (2 * F - a - 1) // 2 + (b - a - 1)

    pid = np.zeros((ND, RING), dtype=np.int32)
    scale = np.zeros((ND, RING), dtype=np.float32)
    for d in range(1, ND + 1):
        for f in range(RING):
            i, j = f, (f + d) % RING
            if i < F and j < F and i != j:
                pid[d - 1, f] = pair_index(min(i, j), max(i, j))
                scale[d - 1, f] = 0.5 if d == ND else 1.0
    return pid.reshape(-1), scale.reshape(-1)


_PID, _SCALE = _pair_perm()


_NW = 32
NEL = B * F * E
_EPW = NEL // _NW
_CHK = 2
_CSZ = _EPW // _CHK


def _sc_gather_body(idx_hbm, tab_hbm, out_hbm, idx_v, dst_v, sem):
    wid = lax.axis_index("s") * 2 + lax.axis_index("c")
    base = wid * _EPW
    for c in range(_CHK):
        off = base + c * _CSZ
        pltpu.sync_copy(idx_hbm.at[pl.ds(off, _CSZ)], idx_v)
        pltpu.async_copy(tab_hbm.at[idx_v], dst_v, sem).wait()
        pltpu.sync_copy(dst_v, out_hbm.at[pl.ds(off, _CSZ)])


def _sc_gather(elem_idx, tab_flat):
    mesh = plsc.VectorSubcoreMesh(core_axis_name="c", subcore_axis_name="s")
    k = pl.kernel(
        _sc_gather_body,
        mesh=mesh,
        compiler_params=pltpu.CompilerParams(use_tc_tiling_on_sc=False),
        out_type=jax.ShapeDtypeStruct((NEL,), jnp.float32),
        scratch_types=[
            pltpu.VMEM((_CSZ,), jnp.int32),
            pltpu.VMEM((_CSZ,), jnp.float32),
            pltpu.SemaphoreType.DMA,
        ],
    )
    return k(elem_idx, tab_flat)


_BT = 256
_GRID = B // _BT


def _tc_body(e_ref, wbig_ref, lb_ref, w1_ref, b1_ref, w2_ref, b2_ref,
             wf_ref, bf_ref, out_ref):
    ep = jnp.concatenate(
        [e_ref[...], jnp.zeros((_BT, KPAD - F * E), jnp.float32)], axis=1)
    acc = jnp.dot(ep, wbig_ref[0:KPAD, :], preferred_element_type=jnp.float32)
    for d in range(1, ND + 1):
        s = E * d
        rot = jnp.concatenate([ep[:, s:], ep[:, :s]], axis=1)
        acc += jnp.dot(ep * rot, wbig_ref[d * KPAD:(d + 1) * KPAD, :],
                       preferred_element_type=jnp.float32)
    x = jnp.maximum(acc + lb_ref[...], 0.0)
    x = jnp.maximum(jnp.dot(x, w1_ref[...],
                            preferred_element_type=jnp.float32) + b1_ref[...], 0.0)
    x = jnp.maximum(jnp.dot(x, w2_ref[...],
                            preferred_element_type=jnp.float32) + b2_ref[...], 0.0)
    z = jnp.dot(x, wf_ref[...], preferred_element_type=jnp.float32) + bf_ref[...]
    out_ref[...] = 1.0 / (1.0 + jnp.exp(-z))


def _prep_wbig(w_z, w_p):
    wz = w_z.reshape(F * E, H0)
    wz = jnp.concatenate([wz, jnp.zeros((KPAD - F * E, H0), jnp.float32)], axis=0)
    wp = jnp.take(w_p, jnp.asarray(_PID), axis=0)
    wp = wp * jnp.asarray(_SCALE)[:, None, None]
    return jnp.concatenate([wz, wp.reshape(ND * KPAD, H0)], axis=0)


def _tc_call(e2, w_big, l_b, W1, b1, W2, b2, Wf, bf):
    const = lambda i: (0, 0)
    return pl.pallas_call(
        _tc_body,
        grid=(_GRID,),
        in_specs=[
            pl.BlockSpec((_BT, F * E), lambda i: (i, 0)),
            pl.BlockSpec(((ND + 1) * KPAD, H0), const),
            pl.BlockSpec((1, H0), const),
            pl.BlockSpec((H0, H1), const),
            pl.BlockSpec((1, H1), const),
            pl.BlockSpec((H1, H2), const),
            pl.BlockSpec((1, H2), const),
            pl.BlockSpec((H2, 1), const),
            pl.BlockSpec((1, 1), const),
        ],
        out_specs=pl.BlockSpec((_BT, 1), lambda i: (i, 0)),
        out_shape=jax.ShapeDtypeStruct((B, 1), jnp.float32),
    )(e2, w_big, l_b.reshape(1, H0), W1, b1.reshape(1, H1),
      W2, b2.reshape(1, H2), Wf, bf.reshape(1, 1))


def kernel(indices, tables, w_z, w_p, l_b, W1, b1, W2, b2, Wf, bf):
    tab_flat = tables.transpose(0, 2, 1).reshape(-1)
    off = (jnp.arange(F, dtype=jnp.int32) * (E * V))[:, None] \
        + (jnp.arange(E, dtype=jnp.int32) * V)[None, :]
    elem_idx = (indices[:, :, None] + off[None, :, :]).reshape(NEL)
    e = _sc_gather(elem_idx, tab_flat)
    e2 = e.reshape(B, F * E)
    w_big = _prep_wbig(w_z, w_p)
    return _tc_call(e2, w_big, l_b, W1, b1, W2, b2, Wf, bf)
